# baseline (speedup 1.0000x reference)
import functools

import jax
import jax.numpy as jnp
from jax import lax
from jax.experimental import pallas as pl
from jax.experimental.pallas import tpu as pltpu
from jax.experimental.pallas import tpu_sc as plsc

H = 56
W = 56
C = 192
NH = 4
HD = C // NH
KS = 7
P = H * W
NC = 2
NS = 16
NW = NC * NS


def _vproj_body(x_ref, w_ref, b_ref, sp_ref, u_ref):
    v = jnp.dot(x_ref[...], w_ref[...], preferred_element_type=jnp.float32)
    u_ref[...] = (v + b_ref[...]) * sp_ref[...]


def _oproj_body(a_ref, w_ref, b_ref, o_ref):
    o_ref[...] = (
        jnp.dot(a_ref[...], w_ref[...], preferred_element_type=jnp.float32)
        + b_ref[...]
    )


_agg_mesh = plsc.VectorSubcoreMesh(core_axis_name="c", subcore_axis_name="s")


@functools.partial(
    pl.kernel,
    out_type=jax.ShapeDtypeStruct((P, C), jnp.float32),
    mesh=_agg_mesh,
    scratch_types=[
        pltpu.VMEM((KS, W, C), jnp.float32),
        pltpu.VMEM((NH * 2744 + 16,), jnp.float32),
        pltpu.VMEM((W, C), jnp.float32),
    ],
)
def _agg(u_hbm, attn_hbm, out_hbm, u_buf, a_buf, o_buf):
    wid = lax.axis_index("s") * NC + lax.axis_index("c")
    NT = KS * KS

    def do_row(h):
        rs = jnp.clip(h - (KS // 2), 0, H - KS)
        pltpu.sync_copy(u_hbm.at[pl.ds(rs, KS)], u_buf)
        for n in range(NH):
            pltpu.sync_copy(
                attn_hbm.at[pl.ds((n * H + h) * (W * NT), W * NT)],
                a_buf.at[pl.ds(n * W * NT, W * NT)],
            )

        def wbody(w, carry):
            cs = jnp.clip(w - (KS // 2), 0, W - KS)
            for n in range(NH):
                base = n * W * NT + w * NT
                avecs = [a_buf[pl.ds(base + 16 * i, 16)] for i in range(4)]
                acc = [jnp.zeros((16,), jnp.float32) for _ in range(3)]
                for k in range(KS):
                    for l in range(KS):
                        col = cs + l
                        idx = k * KS + l
                        av = jnp.broadcast_to(avecs[idx // 16][idx % 16], (16,))
                        for j in range(3):
                            acc[j] = acc[j] + av * u_buf[
                                k, col, pl.ds((n * 3 + j) * 16, 16)
                            ]
                for j in range(3):
                    o_buf[w, pl.ds((n * 3 + j) * 16, 16)] = acc[j]
            return carry

        lax.fori_loop(0, W, wbody, 0)
        pltpu.sync_copy(o_buf, out_hbm.at[pl.ds(h * W, W)])

    def rowbody(i, carry):
        h = wid + i * NW

        @pl.when(h < H)
        def _():
            do_row(h)

        return carry

    lax.fori_loop(0, 2, rowbody, 0)


def kernel(x, attn, imgSp, v_w, v_b, proj_w, proj_b):
    x2d = x.reshape(P, C)
    sp2d = imgSp.reshape(P, 1)
    attn_t = attn.reshape(NH * H * W * KS * KS)

    u = pl.pallas_call(
        _vproj_body,
        out_shape=jax.ShapeDtypeStruct((P, C), jnp.float32),
    )(x2d, v_w.T, v_b.reshape(1, C), sp2d)

    agg = _agg(u.reshape(H, W, C), attn_t)

    out = pl.pallas_call(
        _oproj_body,
        out_shape=jax.ShapeDtypeStruct((P, C), jnp.float32),
    )(agg, proj_w.T, proj_b.reshape(1, C))
    return out.reshape(1, H, W, C)


# speedup vs baseline: 1.1960x; 1.1960x over previous
import functools

import jax
import jax.numpy as jnp
from jax import lax
from jax.experimental import pallas as pl
from jax.experimental.pallas import tpu as pltpu
from jax.experimental.pallas import tpu_sc as plsc

H = 56
W = 56
C = 192
NH = 4
HD = C // NH
KS = 7
P = H * W
NC = 2
NS = 16
NW = NC * NS


def _vproj_body(x_ref, w_ref, b_ref, sp_ref, u_ref):
    v = jnp.dot(x_ref[...], w_ref[...], preferred_element_type=jnp.float32)
    u_ref[...] = (v + b_ref[...]) * sp_ref[...]


def _oproj_body(a_ref, w_ref, b_ref, o_ref):
    o_ref[...] = (
        jnp.dot(a_ref[...], w_ref[...], preferred_element_type=jnp.float32)
        + b_ref[...]
    )


_agg_mesh = plsc.VectorSubcoreMesh(core_axis_name="c", subcore_axis_name="s")


@functools.partial(
    pl.kernel,
    out_type=jax.ShapeDtypeStruct((P, C), jnp.float32),
    mesh=_agg_mesh,
    scratch_types=[
        pltpu.VMEM((KS, W, C), jnp.float32),
        pltpu.VMEM((W, 208), jnp.float32),
        pltpu.VMEM((W, C), jnp.float32),
    ],
)
def _agg(u_hbm, attn_hbm, out_hbm, u_buf, a_buf, o_buf):
    wid = lax.axis_index("s") * NC + lax.axis_index("c")

    def do_row(h):
        rs = jnp.clip(h - (KS // 2), 0, H - KS)
        pltpu.sync_copy(u_hbm.at[pl.ds(rs, KS)], u_buf)
        pltpu.sync_copy(attn_hbm.at[h], a_buf)

        def wbody(i, carry):
            w = i + 48 * (i // 4)
            cs = jnp.clip(w - (KS // 2), 0, W - KS)
            for n in range(NH):
                lo = (n * KS * KS) // 16
                sh = n * KS * KS - lo * 16
                avecs = [a_buf[w, pl.ds((lo + i2) * 16, 16)] for i2 in range(4)]
                acc = [jnp.zeros((16,), jnp.float32) for _ in range(3)]
                for k in range(KS):
                    for l in range(KS):
                        col = cs + l
                        t = sh + k * KS + l
                        av = jnp.broadcast_to(avecs[t // 16][t % 16], (16,))
                        for j in range(3):
                            acc[j] = acc[j] + av * u_buf[
                                k, col, pl.ds((n * 3 + j) * 16, 16)
                            ]
                for j in range(3):
                    o_buf[w, pl.ds((n * 3 + j) * 16, 16)] = acc[j]
            return carry

        def pairbody(m, carry):
            w0 = 2 * m
            cs0 = w0 - (KS // 2)
            for n in range(NH):
                lo = (n * KS * KS) // 16
                sh = n * KS * KS - lo * 16
                a0 = [a_buf[w0, pl.ds((lo + i2) * 16, 16)] for i2 in range(4)]
                a1 = [
                    a_buf[w0 + 1, pl.ds((lo + i2) * 16, 16)] for i2 in range(4)
                ]
                acc0 = [jnp.zeros((16,), jnp.float32) for _ in range(3)]
                acc1 = [jnp.zeros((16,), jnp.float32) for _ in range(3)]
                for k in range(KS):
                    cv = [
                        [
                            u_buf[k, cs0 + c, pl.ds((n * 3 + j) * 16, 16)]
                            for j in range(3)
                        ]
                        for c in range(KS + 1)
                    ]
                    for l in range(KS):
                        t = sh + k * KS + l
                        av0 = jnp.broadcast_to(a0[t // 16][t % 16], (16,))
                        av1 = jnp.broadcast_to(a1[t // 16][t % 16], (16,))
                        for j in range(3):
                            acc0[j] = acc0[j] + av0 * cv[l][j]
                            acc1[j] = acc1[j] + av1 * cv[l + 1][j]
                for j in range(3):
                    o_buf[w0, pl.ds((n * 3 + j) * 16, 16)] = acc0[j]
                    o_buf[w0 + 1, pl.ds((n * 3 + j) * 16, 16)] = acc1[j]
            return carry

        lax.fori_loop(0, 8, wbody, 0)
        lax.fori_loop(2, 26, pairbody, 0)
        pltpu.sync_copy(o_buf, out_hbm.at[pl.ds(h * W, W)])

    def rowbody(i, carry):
        h = wid + i * NW

        @pl.when(h < H)
        def _():
            do_row(h)

        return carry

    lax.fori_loop(0, 2, rowbody, 0)


def kernel(x, attn, imgSp, v_w, v_b, proj_w, proj_b):
    x2d = x.reshape(P, C)
    sp2d = imgSp.reshape(P, 1)
    attn_t = attn.reshape(NH, H, W, KS * KS).transpose(1, 2, 0, 3)
    attn_t = attn_t.reshape(H, W, NH * KS * KS)
    attn_t = jnp.pad(attn_t, ((0, 0), (0, 0), (0, 208 - NH * KS * KS)))

    u = pl.pallas_call(
        _vproj_body,
        out_shape=jax.ShapeDtypeStruct((P, C), jnp.float32),
    )(x2d, v_w.T, v_b.reshape(1, C), sp2d)

    agg = _agg(u.reshape(H, W, C), attn_t)

    out = pl.pallas_call(
        _oproj_body,
        out_shape=jax.ShapeDtypeStruct((P, C), jnp.float32),
    )(agg, proj_w.T, proj_b.reshape(1, C))
    return out.reshape(1, H, W, C)


# speedup vs baseline: 1.2086x; 1.0105x over previous
import functools

import jax
import jax.numpy as jnp
from jax import lax
from jax.experimental import pallas as pl
from jax.experimental.pallas import tpu as pltpu
from jax.experimental.pallas import tpu_sc as plsc

H = 56
W = 56
C = 192
NH = 4
HD = C // NH
KS = 7
P = H * W
NC = 2
NS = 16
NW = NC * NS


_DN_T = (((1,), (1,)), ((), ()))


def _vproj_body(x_ref, w_ref, b_ref, sp_ref, u_ref):
    v = lax.dot_general(
        x_ref[...], w_ref[...], _DN_T, preferred_element_type=jnp.float32
    )
    u_ref[...] = (v + b_ref[...]) * sp_ref[...]


def _oproj_body(a_ref, w_ref, b_ref, o_ref):
    o_ref[...] = (
        lax.dot_general(
            a_ref[...], w_ref[...], _DN_T, preferred_element_type=jnp.float32
        )
        + b_ref[...]
    )


_agg_mesh = plsc.VectorSubcoreMesh(core_axis_name="c", subcore_axis_name="s")


@functools.partial(
    pl.kernel,
    out_type=jax.ShapeDtypeStruct((P, C), jnp.float32),
    mesh=_agg_mesh,
    scratch_types=[
        pltpu.VMEM((KS, W, C), jnp.float32),
        pltpu.VMEM((W, 208), jnp.float32),
        pltpu.VMEM((W, C), jnp.float32),
        pltpu.SemaphoreType.DMA,
        pltpu.SemaphoreType.DMA,
    ],
)
def _agg(u_hbm, attn_hbm, out_hbm, u_buf, a_buf, o_buf, usem, asem):
    wid = lax.axis_index("s") * NC + lax.axis_index("c")

    def do_row(h):
        rs = jnp.clip(h - (KS // 2), 0, H - KS)
        cu = pltpu.async_copy(u_hbm.at[pl.ds(rs, KS)], u_buf, usem)
        ca = pltpu.async_copy(attn_hbm.at[h], a_buf, asem)
        cu.wait()
        ca.wait()

        def wbody(i, carry):
            w = i + 48 * (i // 4)
            cs = jnp.clip(w - (KS // 2), 0, W - KS)
            for n in range(NH):
                lo = (n * KS * KS) // 16
                sh = n * KS * KS - lo * 16
                avecs = [a_buf[w, pl.ds((lo + i2) * 16, 16)] for i2 in range(4)]
                acc = [jnp.zeros((16,), jnp.float32) for _ in range(3)]
                for k in range(KS):
                    for l in range(KS):
                        col = cs + l
                        t = sh + k * KS + l
                        av = jnp.broadcast_to(avecs[t // 16][t % 16], (16,))
                        for j in range(3):
                            acc[j] = acc[j] + av * u_buf[
                                k, col, pl.ds((n * 3 + j) * 16, 16)
                            ]
                for j in range(3):
                    o_buf[w, pl.ds((n * 3 + j) * 16, 16)] = acc[j]
            return carry

        def pairbody(m, carry):
            w0 = 2 * m
            cs0 = w0 - (KS // 2)
            for n in range(NH):
                lo = (n * KS * KS) // 16
                sh = n * KS * KS - lo * 16
                a0 = [a_buf[w0, pl.ds((lo + i2) * 16, 16)] for i2 in range(4)]
                a1 = [
                    a_buf[w0 + 1, pl.ds((lo + i2) * 16, 16)] for i2 in range(4)
                ]
                ac = [
                    [jnp.zeros((16,), jnp.float32) for _ in range(3)]
                    for _ in range(4)
                ]
                for k in range(KS):
                    cv = [
                        [
                            u_buf[k, cs0 + c, pl.ds((n * 3 + j) * 16, 16)]
                            for j in range(3)
                        ]
                        for c in range(KS + 1)
                    ]
                    for l in range(KS):
                        t = sh + k * KS + l
                        av0 = jnp.broadcast_to(a0[t // 16][t % 16], (16,))
                        av1 = jnp.broadcast_to(a1[t // 16][t % 16], (16,))
                        b = l & 1
                        for j in range(3):
                            ac[b][j] = ac[b][j] + av0 * cv[l][j]
                            ac[2 + b][j] = ac[2 + b][j] + av1 * cv[l + 1][j]
                for j in range(3):
                    o_buf[w0, pl.ds((n * 3 + j) * 16, 16)] = ac[0][j] + ac[1][j]
                    o_buf[w0 + 1, pl.ds((n * 3 + j) * 16, 16)] = (
                        ac[2][j] + ac[3][j]
                    )
            return carry

        lax.fori_loop(0, 8, wbody, 0)
        lax.fori_loop(2, 26, pairbody, 0)
        pltpu.sync_copy(o_buf, out_hbm.at[pl.ds(h * W, W)])

    def rowbody(i, carry):
        h = wid + i * NW

        @pl.when(h < H)
        def _():
            do_row(h)

        return carry

    lax.fori_loop(0, 2, rowbody, 0)


def kernel(x, attn, imgSp, v_w, v_b, proj_w, proj_b):
    x2d = x.reshape(P, C)
    sp2d = imgSp.reshape(P, 1)
    attn_t = attn.reshape(NH, H, W, KS * KS).transpose(1, 2, 0, 3)
    attn_t = attn_t.reshape(H, W, NH * KS * KS)
    attn_t = jnp.pad(attn_t, ((0, 0), (0, 0), (0, 208 - NH * KS * KS)))

    u = pl.pallas_call(
        _vproj_body,
        out_shape=jax.ShapeDtypeStruct((P, C), jnp.float32),
    )(x2d, v_w, v_b.reshape(1, C), sp2d)

    agg = _agg(u.reshape(H, W, C), attn_t)

    out = pl.pallas_call(
        _oproj_body,
        out_shape=jax.ShapeDtypeStruct((P, C), jnp.float32),
    )(agg, proj_w, proj_b.reshape(1, C))
    return out.reshape(1, H, W, C)


# speedup vs baseline: 1.2154x; 1.0057x over previous
import functools

import jax
import jax.numpy as jnp
from jax import lax
from jax.experimental import pallas as pl
from jax.experimental.pallas import tpu as pltpu
from jax.experimental.pallas import tpu_sc as plsc

H = 56
W = 56
C = 192
NH = 4
HD = C // NH
KS = 7
P = H * W
NC = 2
NS = 16
NW = NC * NS


_DN_T = (((1,), (1,)), ((), ()))


def _vproj_body(x_ref, w_ref, b_ref, sp_ref, u_ref):
    v = lax.dot_general(
        x_ref[...], w_ref[...], _DN_T, preferred_element_type=jnp.float32
    )
    u_ref[...] = (v + b_ref[...]) * sp_ref[...]


def _oproj_body(a_ref, w_ref, b_ref, o_ref):
    o_ref[...] = (
        lax.dot_general(
            a_ref[...], w_ref[...], _DN_T, preferred_element_type=jnp.float32
        )
        + b_ref[...]
    )


_agg_mesh = plsc.VectorSubcoreMesh(core_axis_name="c", subcore_axis_name="s")


@functools.partial(
    pl.kernel,
    out_type=jax.ShapeDtypeStruct((P, C), jnp.float32),
    mesh=_agg_mesh,
    scratch_types=[
        pltpu.VMEM((KS, W, C), jnp.float32),
        pltpu.VMEM((W, 208), jnp.float32),
        pltpu.VMEM((W, C), jnp.float32),
        pltpu.SemaphoreType.DMA,
        pltpu.SemaphoreType.DMA,
    ],
)
def _agg(u_hbm, attn_hbm, out_hbm, u_buf, a_buf, o_buf, usem, asem):
    wid = lax.axis_index("s") * NC + lax.axis_index("c")

    def do_row(h):
        rs = jnp.clip(h - (KS // 2), 0, H - KS)
        cu = pltpu.async_copy(u_hbm.at[pl.ds(rs, KS)], u_buf, usem)
        ca = pltpu.async_copy(attn_hbm.at[h], a_buf, asem)
        cu.wait()
        ca.wait()

        def inner(w0, cs0, d1):
            for n in range(NH):
                lo = (n * KS * KS) // 16
                sh = n * KS * KS - lo * 16
                a0 = [a_buf[w0, pl.ds((lo + i2) * 16, 16)] for i2 in range(4)]
                a1 = [
                    a_buf[w0 + 1, pl.ds((lo + i2) * 16, 16)] for i2 in range(4)
                ]
                ac = [
                    [jnp.zeros((16,), jnp.float32) for _ in range(3)]
                    for _ in range(4)
                ]
                for k in range(KS):
                    cv = [
                        [
                            u_buf[k, cs0 + c, pl.ds((n * 3 + j) * 16, 16)]
                            for j in range(3)
                        ]
                        for c in range(KS + d1)
                    ]
                    for l in range(KS):
                        t = sh + k * KS + l
                        av0 = jnp.broadcast_to(a0[t // 16][t % 16], (16,))
                        av1 = jnp.broadcast_to(a1[t // 16][t % 16], (16,))
                        b = l & 1
                        for j in range(3):
                            ac[b][j] = ac[b][j] + av0 * cv[l][j]
                            ac[2 + b][j] = ac[2 + b][j] + av1 * cv[l + d1][j]
                for j in range(3):
                    o_buf[w0, pl.ds((n * 3 + j) * 16, 16)] = ac[0][j] + ac[1][j]
                    o_buf[w0 + 1, pl.ds((n * 3 + j) * 16, 16)] = (
                        ac[2][j] + ac[3][j]
                    )

        def wbody(i, carry):
            m2 = i + 24 * (i // 2)
            w0 = 2 * m2
            cs0 = jnp.clip(w0 - (KS // 2), 0, W - KS)
            inner(w0, cs0, 0)
            return carry

        def pairbody(m, carry):
            w0 = 2 * m
            inner(w0, w0 - (KS // 2), 1)
            return carry

        lax.fori_loop(0, 4, wbody, 0)
        lax.fori_loop(2, 26, pairbody, 0)
        pltpu.sync_copy(o_buf, out_hbm.at[pl.ds(h * W, W)])

    def rowbody(i, carry):
        h = wid + i * NW

        @pl.when(h < H)
        def _():
            do_row(h)

        return carry

    lax.fori_loop(0, 2, rowbody, 0)


def kernel(x, attn, imgSp, v_w, v_b, proj_w, proj_b):
    x2d = x.reshape(P, C)
    sp2d = imgSp.reshape(P, 1)
    attn_t = attn.reshape(NH, H, W, KS * KS).transpose(1, 2, 0, 3)
    attn_t = attn_t.reshape(H, W, NH * KS * KS)
    attn_t = jnp.pad(attn_t, ((0, 0), (0, 0), (0, 208 - NH * KS * KS)))

    u = pl.pallas_call(
        _vproj_body,
        out_shape=jax.ShapeDtypeStruct((P, C), jnp.float32),
    )(x2d, v_w, v_b.reshape(1, C), sp2d)

    agg = _agg(u.reshape(H, W, C), attn_t)

    out = pl.pallas_call(
        _oproj_body,
        out_shape=jax.ShapeDtypeStruct((P, C), jnp.float32),
    )(agg, proj_w, proj_b.reshape(1, C))
    return out.reshape(1, H, W, C)
